# Initial kernel scaffold; baseline (speedup 1.0000x reference)
import jax
import jax.numpy as jnp
from jax.experimental import pallas as pl


def kernel(x, cls_attention_map):
    raise NotImplementedError("write your pallas kernel here")


# speedup vs baseline: 4.0642x; 4.0642x over previous
import functools
import numpy as np
import jax
import jax.numpy as jnp
from jax.experimental import pallas as pl
from jax.experimental.pallas import tpu as pltpu

_DB4_DEC_LO = np.array([
    -0.010597401785069032, 0.0328830116668852, 0.030841381835560764,
    -0.18703481171909309, -0.027983769416859854, 0.6308807679298589,
    0.7148465705529157, 0.2303778133088965], dtype=np.float64)
_H0 = _DB4_DEC_LO[::-1].copy()


def _dwt_band_matrix(n_in: int) -> np.ndarray:
    n_out = (n_in + 7) // 2
    W = np.zeros((n_out, n_in), np.float64)
    for n in range(n_out):
        for k in range(8):
            m = 2 * n + k - 6
            if 0 <= m < n_in:
                W[n, m] = _H0[k]
    return W


_P = 1024
_W1 = _dwt_band_matrix(_P)
_W2 = _dwt_band_matrix(_W1.shape[0])
_W3 = _dwt_band_matrix(_W2.shape[0])
_LEN1, _LEN2, _LEN3 = _W1.shape[0], _W2.shape[0], _W3.shape[0]
_OUT_PAD = 520


def _padded(M: np.ndarray) -> np.ndarray:
    out = np.zeros((_OUT_PAD, _P), np.float32)
    out[:M.shape[0]] = M.astype(np.float32)
    return out


_MSTACK = np.stack([_padded(_W1), _padded(_W2 @ _W1),
                    _padded(_W3 @ _W2 @ _W1)])


def _routing_body(att_ref, levels_ref, mask_ref):
    B = att_ref.shape[0]
    a = att_ref[:]
    terms = a * jnp.log2(a + 1e-9)
    ent = -jnp.sum(terms, axis=1, keepdims=True)
    ii = jax.lax.broadcasted_iota(jnp.int32, (B, B), 0)
    jj = jax.lax.broadcasted_iota(jnp.int32, (B, B), 1)
    eyef = (ii == jj).astype(jnp.float32)
    ej = jnp.dot(jnp.ones((B, B), jnp.float32), eyef * ent,
                 preferred_element_type=jnp.float32,
                 precision=jax.lax.Precision.HIGHEST)
    lt = (ej < ent) | ((ej == ent) & (jj < ii))
    rank = jnp.sum(lt.astype(jnp.float32), axis=1, keepdims=True)

    def order_stat(k):
        return jnp.sum(jnp.where(rank == k, ent, 0.0))

    i25 = 0.25 * (B - 1)
    i50 = 0.5 * (B - 1)
    f25, c25 = int(np.floor(i25)), int(np.ceil(i25))
    f50, c50 = int(np.floor(i50)), int(np.ceil(i50))
    q25 = order_stat(f25) + (i25 - f25) * (order_stat(c25) - order_stat(f25))
    q50 = order_stat(f50) + (i50 - f50) * (order_stat(c50) - order_stat(f50))
    levels = (3 - (ent > q25).astype(jnp.int32)
                - (ent > q50).astype(jnp.int32))
    lengths = jnp.where(levels == 1, _LEN1,
                        jnp.where(levels == 2, _LEN2, _LEN3))
    levels_ref[:] = levels
    cols = jax.lax.broadcasted_iota(jnp.int32, (B, _OUT_PAD), 1)
    mask_ref[:] = (cols < lengths).astype(jnp.int32)


def _dwt_body(lv_ref, m_ref, x_ref, out_ref):
    b = pl.program_id(0)
    m = m_ref[lv_ref[b]]
    out_ref[0] = jax.lax.dot_general(
        m, x_ref[0], (((1,), (0,)), ((), ())),
        preferred_element_type=jnp.float32)


def kernel(x, cls_attention_map):
    B, N, D = x.shape
    patches = x[:, 1:, :]

    levels2d, mask_i = pl.pallas_call(
        _routing_body,
        out_shape=(
            jax.ShapeDtypeStruct((B, 1), jnp.int32),
            jax.ShapeDtypeStruct((B, _OUT_PAD), jnp.int32),
        ),
    )(cls_attention_map)
    levels0 = (levels2d - 1).reshape(B)

    mstack = jnp.asarray(_MSTACK)
    out = pl.pallas_call(
        _dwt_body,
        grid=(B,),
        in_specs=[
            pl.BlockSpec(memory_space=pltpu.SMEM),
            pl.BlockSpec((3, _OUT_PAD, _P), lambda b: (0, 0, 0)),
            pl.BlockSpec((1, _P, D), lambda b: (b, 0, 0)),
        ],
        out_specs=pl.BlockSpec((1, _OUT_PAD, D), lambda b: (b, 0, 0)),
        out_shape=jax.ShapeDtypeStruct((B, _OUT_PAD, D), jnp.float32),
    )(levels0, mstack, patches)

    final_x = jnp.concatenate([x[:, :1, :], out[:, :_LEN1, :]], axis=1)
    attention_mask = jnp.concatenate(
        [jnp.ones((B, 1), bool), mask_i[:, :_LEN1].astype(bool)], axis=1)
    return (final_x, attention_mask)


# speedup vs baseline: 6.7532x; 1.6616x over previous
import functools
import numpy as np
import jax
import jax.numpy as jnp
from jax.experimental import pallas as pl
from jax.experimental.pallas import tpu as pltpu

_DB4_DEC_LO = np.array([
    -0.010597401785069032, 0.0328830116668852, 0.030841381835560764,
    -0.18703481171909309, -0.027983769416859854, 0.6308807679298589,
    0.7148465705529157, 0.2303778133088965], dtype=np.float64)
_H0 = _DB4_DEC_LO[::-1].copy()


def _dwt_band_matrix(n_in: int) -> np.ndarray:
    n_out = (n_in + 7) // 2
    W = np.zeros((n_out, n_in), np.float64)
    for n in range(n_out):
        for k in range(8):
            m = 2 * n + k - 6
            if 0 <= m < n_in:
                W[n, m] = _H0[k]
    return W


_P = 1024
_W1 = _dwt_band_matrix(_P)
_W2 = _dwt_band_matrix(_W1.shape[0])
_W3 = _dwt_band_matrix(_W2.shape[0])
_LEN1, _LEN2, _LEN3 = _W1.shape[0], _W2.shape[0], _W3.shape[0]
_OUT = _LEN1 + 1
_OUT_PAD = 520


def _padded(M: np.ndarray) -> np.ndarray:
    out = np.zeros((_OUT_PAD, _P + 1), np.float32)
    out[0, 0] = 1.0
    out[1:1 + M.shape[0], 1:] = M.astype(np.float32)
    return out


_MSTACK = np.stack([_padded(_W1), _padded(_W2 @ _W1),
                    _padded(_W3 @ _W2 @ _W1)])


def _routing_body(att_ref, levels_ref, mask_ref):
    B = att_ref.shape[0]
    a = att_ref[:]
    terms = a * jnp.log2(a + 1e-9)
    ent = -jnp.sum(terms, axis=1, keepdims=True)
    ii = jax.lax.broadcasted_iota(jnp.int32, (B, B), 0)
    jj = jax.lax.broadcasted_iota(jnp.int32, (B, B), 1)
    eyef = (ii == jj).astype(jnp.float32)
    ej = jnp.dot(jnp.ones((B, B), jnp.float32), eyef * ent,
                 preferred_element_type=jnp.float32,
                 precision=jax.lax.Precision.HIGHEST)
    lt = (ej < ent) | ((ej == ent) & (jj < ii))
    rank = jnp.sum(lt.astype(jnp.float32), axis=1, keepdims=True)

    def order_stat(k):
        return jnp.sum(jnp.where(rank == k, ent, 0.0))

    i25 = 0.25 * (B - 1)
    i50 = 0.5 * (B - 1)
    f25, c25 = int(np.floor(i25)), int(np.ceil(i25))
    f50, c50 = int(np.floor(i50)), int(np.ceil(i50))
    q25 = order_stat(f25) + (i25 - f25) * (order_stat(c25) - order_stat(f25))
    q50 = order_stat(f50) + (i50 - f50) * (order_stat(c50) - order_stat(f50))
    levels = (3 - (ent > q25).astype(jnp.int32)
                - (ent > q50).astype(jnp.int32))
    lengths = jnp.where(levels == 1, _LEN1,
                        jnp.where(levels == 2, _LEN2, _LEN3))
    levels_ref[:] = levels
    cols = jax.lax.broadcasted_iota(jnp.int32, (B, _OUT), 1)
    mask_ref[:] = cols < (lengths + 1)


def _dwt_body(lv_ref, m_ref, x_ref, out_ref):
    b = pl.program_id(0)
    m = m_ref[lv_ref[b]]
    res = jax.lax.dot_general(
        m, x_ref[0], (((1,), (0,)), ((), ())),
        preferred_element_type=jnp.float32)
    out_ref[0] = res[:_OUT]


def kernel(x, cls_attention_map):
    B, N, D = x.shape

    levels2d, attention_mask = pl.pallas_call(
        _routing_body,
        out_shape=(
            jax.ShapeDtypeStruct((B, 1), jnp.int32),
            jax.ShapeDtypeStruct((B, _OUT), jnp.bool_),
        ),
    )(cls_attention_map)
    levels0 = (levels2d - 1).reshape(B)

    mstack = jnp.asarray(_MSTACK)
    final_x = pl.pallas_call(
        _dwt_body,
        grid=(B,),
        in_specs=[
            pl.BlockSpec(memory_space=pltpu.SMEM),
            pl.BlockSpec((3, _OUT_PAD, _P + 1), lambda b: (0, 0, 0)),
            pl.BlockSpec((1, N, D), lambda b: (b, 0, 0)),
        ],
        out_specs=pl.BlockSpec((1, _OUT, D), lambda b: (b, 0, 0)),
        out_shape=jax.ShapeDtypeStruct((B, _OUT, D), jnp.float32),
    )(levels0, mstack, x)

    return (final_x, attention_mask)
